# Initial kernel scaffold; baseline (speedup 1.0000x reference)
import jax
import jax.numpy as jnp
from jax.experimental import pallas as pl


def kernel(x, edge_index, batch, params):
    raise NotImplementedError("write your pallas kernel here")


# speedup vs baseline: 1.8732x; 1.8732x over previous
import functools

import numpy as np
import jax
import jax.numpy as jnp
from jax import lax
from jax.experimental import pallas as pl
from jax.experimental.pallas import tpu as pltpu
from jax.experimental.pallas import tpu_sc as plsc

N = 10000
E = 320000
D = 128
H = 128
LAT = 64
HD = 256
B = 200
NMAX = 50
NPAIR = NMAX * (NMAX - 1) // 2
BP = 256
PPAD = 1280
APAD = 2560

NC = 2
NS = 16
NW = NC * NS
CK = 128
CKT = 16
HALF = E // 2
NPAD = 10240
SLAB = NPAD // NS
SLABW = 384
ACCR = 6400
ACCOFF = NPAD - ACCR

RB = 1000
GN = N // RB

_PREC = None
_PREC_EXACT = jax.lax.Precision.HIGHEST


def _seg_sum_sc(h, src_sorted, dst_sorted):
    mesh = plsc.VectorSubcoreMesh(core_axis_name="c", subcore_axis_name="s")

    @functools.partial(
        pl.kernel,
        out_type=jax.ShapeDtypeStruct((2, NPAD, D), jnp.float32),
        mesh=mesh,
        scratch_types=[
            pltpu.VMEM((CK,), jnp.int32),
            pltpu.VMEM((CK,), jnp.int32),
            pltpu.VMEM((CK, D), jnp.float32),
            pltpu.VMEM((CKT,), jnp.int32),
            pltpu.VMEM((CKT,), jnp.int32),
            pltpu.VMEM((CKT, D), jnp.float32),
            pltpu.VMEM((SLABW, D), jnp.float32),
            pltpu.VMEM((CK,), jnp.int32),
            pltpu.VMEM_SHARED((ACCR, D), jnp.float32),
            pltpu.SemaphoreType.DMA,
        ],
    )
    def seg(h_hbm, src_hbm, dst_hbm, out_hbm, src_v, dst_v, rows_v,
            st_v, dt_v, rowt_v, slab_v, idxf_v, acc, sem):
        c = lax.axis_index("c")
        s = lax.axis_index("s")
        arow0 = s * (ACCR // NS)

        zero16 = jnp.zeros((16,), jnp.float32)

        def zrow(i, carry):
            for cc in range(D // 16):
                rows_v[i, pl.ds(cc * 16, 16)] = zero16
            return carry

        lax.fori_loop(0, CK, zrow, 0, unroll=False)

        def zslab(j, carry):
            pltpu.sync_copy(rows_v, acc.at[pl.ds(arow0 + j * CK, CK)])
            return carry

        lax.fori_loop(0, 3, zslab, 0, unroll=False)
        pltpu.sync_copy(rows_v.at[pl.ds(0, 16)],
                        acc.at[pl.ds(arow0 + 3 * CK, 16)])

        def zs(i, carry):
            for cc in range(D // 16):
                slab_v[i, pl.ds(cc * 16, 16)] = zero16
            return carry

        lax.fori_loop(0, SLABW, zs, 0, unroll=False)
        plsc.subcore_barrier()

        length = jnp.where(s < 11, 10080, jnp.where(s < 15, 9840, 9760))
        cum = jnp.where(s < 11, s * 10080,
                        jnp.where(s < 15, 110880 + (s - 11) * 9840, 150240))
        base0 = c * HALF + cum
        nb = length // CK
        tb = base0 + nb * CK
        nt = (length - nb * CK) // CKT

        pltpu.sync_copy(dst_hbm.at[pl.ds(base0, CK)], dst_v)
        first = dst_v[pl.ds(0, 16)][0]

        def batch(j, prev):
            b = base0 + j * CK
            pltpu.sync_copy(src_hbm.at[pl.ds(b, CK)], src_v)
            pltpu.sync_copy(dst_hbm.at[pl.ds(b, CK)], dst_v)
            pltpu.async_copy(h_hbm.at[src_v], rows_v, sem).wait()

            def group(g, prev):
                dvec = dst_v[pl.ds(g * 16, 16)]
                for e in range(16):
                    dg = dvec[e]
                    dl = dg - first
                    nr = dg != prev
                    for cc in range(D // 16):
                        v = rows_v[g * 16 + e, pl.ds(cc * 16, 16)]
                        pv = slab_v[dl, pl.ds(cc * 16, 16)]
                        slab_v[dl, pl.ds(cc * 16, 16)] = jnp.where(
                            nr, v, pv + v)
                    prev = dg
                return prev

            return lax.fori_loop(0, CK // 16, group, prev, unroll=False)

        prev = lax.fori_loop(0, nb, batch, jnp.int32(-1), unroll=False)

        def batchT(j, prev):
            b = tb + j * CKT
            pltpu.sync_copy(src_hbm.at[pl.ds(b, CKT)], st_v)
            pltpu.sync_copy(dst_hbm.at[pl.ds(b, CKT)], dt_v)
            pltpu.async_copy(h_hbm.at[st_v], rowt_v, sem).wait()

            dvec = dt_v[pl.ds(0, 16)]
            for e in range(16):
                dg = dvec[e]
                dl = dg - first
                nr = dg != prev
                for cc in range(D // 16):
                    v = rowt_v[e, pl.ds(cc * 16, 16)]
                    pv = slab_v[dl, pl.ds(cc * 16, 16)]
                    slab_v[dl, pl.ds(cc * 16, 16)] = jnp.where(nr, v, pv + v)
                prev = dg
            return prev

        lax.fori_loop(0, nt, batchT, prev, unroll=False)

        iota16 = lax.iota(jnp.int32, 16)
        fbase = first - c * ACCOFF

        def flush(k, carry):
            for g in range(D // 16):
                idxf_v[pl.ds(g * 16, 16)] = jnp.clip(
                    fbase + k * CK + g * 16 + iota16, 0, ACCR - 1)
            pltpu.sync_copy(slab_v.at[pl.ds(k * CK, CK)],
                            acc.at[idxf_v], add=True)
            return carry

        lax.fori_loop(0, SLABW // CK, flush, 0, unroll=False)
        plsc.subcore_barrier()
        pltpu.sync_copy(acc.at[pl.ds(arow0, ACCR // NS)],
                        out_hbm.at[c, pl.ds(c * ACCOFF + arow0, ACCR // NS)])
        lax.fori_loop(0, CK, zrow, 0, unroll=False)
        zbase = jnp.where(c == 0, ACCR, 0)

        def zfill(j, carry):
            g = s + NS * j
            pltpu.sync_copy(rows_v.at[pl.ds(0, 16)],
                            out_hbm.at[c, pl.ds(zbase + 16 * g, 16)])
            return carry

        lax.fori_loop(0, (NPAD - ACCR) // 16 // NS, zfill, 0, unroll=False)

    return seg(h, src_sorted, dst_sorted)


def _fc0(x, W, b):
    def body(x_ref, w_ref, b_ref, o_ref):
        o_ref[...] = (
            jnp.dot(x_ref[...], w_ref[...], precision=_PREC,
                    preferred_element_type=jnp.float32) + b_ref[...])

    return pl.pallas_call(
        body,
        grid=(GN,),
        in_specs=[pl.BlockSpec((RB, D), lambda i: (i, 0)),
                  pl.BlockSpec((D, H), lambda i: (0, 0)),
                  pl.BlockSpec((1, H), lambda i: (0, 0))],
        out_specs=pl.BlockSpec((RB, H), lambda i: (i, 0)),
        out_shape=jax.ShapeDtypeStruct((N, H), jnp.float32),
    )(x, W, b)


def _gin_mlp(h, parts, W1, b1, a1, s1, t1, W2, b2, a2, s2, t2):
    def body(h_ref, p0_ref, p1_ref, w1, b1r, a1r, s1r, t1r,
             w2, b2r, a2r, s2r, t2r, o_ref):
        m = h_ref[...] + p0_ref[0] + p1_ref[0]
        v = jnp.dot(m, w1[...], precision=_PREC,
                    preferred_element_type=jnp.float32) + b1r[...]
        v = jnp.where(v >= 0, v, a1r[...] * v)
        v = v * s1r[...] + t1r[...]
        w = jnp.dot(v, w2[...], precision=_PREC,
                    preferred_element_type=jnp.float32) + b2r[...]
        w = jnp.where(w >= 0, w, a2r[...] * w)
        o_ref[...] = w * s2r[...] + t2r[...]

    def vec():
        return pl.BlockSpec((1, H), lambda i: (0, 0))

    return pl.pallas_call(
        body,
        grid=(GN,),
        in_specs=[pl.BlockSpec((RB, H), lambda i: (i, 0)),
                  pl.BlockSpec((1, RB, H), lambda i: (0, i, 0)),
                  pl.BlockSpec((1, RB, H), lambda i: (1, i, 0)),
                  pl.BlockSpec((H, H), lambda i: (0, 0)),
                  vec(), vec(), vec(), vec(),
                  pl.BlockSpec((H, H), lambda i: (0, 0)),
                  vec(), vec(), vec(), vec()],
        out_specs=pl.BlockSpec((RB, H), lambda i: (i, 0)),
        out_shape=jax.ShapeDtypeStruct((N, H), jnp.float32),
    )(h, parts, parts, W1, b1, a1, s1, t1, W2, b2, a2, s2, t2)


def _pool_head(h, bids, bn_s, bn_b, fc_W, fc_b, mu_W, mu_b, lv_W, lv_b):
    def body(h_ref, id_ref, bs, bb, fw, fb, mw, mb, lw, lb,
             mu_o, lv_o, acc):
        i = pl.program_id(0)

        @pl.when(i == 0)
        def _():
            acc[...] = jnp.zeros_like(acc)

        ids = id_ref[0]
        ohT = (lax.broadcasted_iota(jnp.int32, (BP, RB), 0) == ids
               ).astype(jnp.float32)
        acc[...] += jnp.dot(ohT, h_ref[...], precision=_PREC_EXACT,
                            preferred_element_type=jnp.float32)

        @pl.when(i == GN - 1)
        def _():
            pooled = acc[...] * bs[...] + bb[...]
            xg = jnp.dot(pooled, fw[...], precision=_PREC,
                         preferred_element_type=jnp.float32) + fb[...]
            mu_o[...] = jnp.dot(xg, mw[...], precision=_PREC,
                                preferred_element_type=jnp.float32) + mb[...]
            lv_o[...] = jnp.dot(xg, lw[...], precision=_PREC,
                                preferred_element_type=jnp.float32) + lb[...]

    def vec(w):
        return pl.BlockSpec((1, w), lambda i: (0, 0))

    return pl.pallas_call(
        body,
        grid=(GN,),
        in_specs=[pl.BlockSpec((RB, H), lambda i: (i, 0)),
                  pl.BlockSpec((1, 1, RB), lambda i: (i, 0, 0)),
                  vec(H), vec(H),
                  pl.BlockSpec((H, H), lambda i: (0, 0)), vec(H),
                  pl.BlockSpec((H, LAT), lambda i: (0, 0)), vec(LAT),
                  pl.BlockSpec((H, LAT), lambda i: (0, 0)), vec(LAT)],
        out_specs=[pl.BlockSpec((BP, LAT), lambda i: (0, 0)),
                   pl.BlockSpec((BP, LAT), lambda i: (0, 0))],
        out_shape=[jax.ShapeDtypeStruct((BP, LAT), jnp.float32),
                   jax.ShapeDtypeStruct((BP, LAT), jnp.float32)],
        scratch_shapes=[pltpu.VMEM((BP, H), jnp.float32)],
    )(h, bids, bn_s, bn_b, fc_W, fc_b, mu_W, mu_b, lv_W, lv_b)


def _decoder(z, d0W, d0b, d1W, d1b, W2e, b2e, W2f, b2f, g0, g1, Gm, av):
    def body(z_ref, w0, b0, w1, b1r, we, be, wf, bf, g0r, g1r, gmr, ar,
             o_ref):
        dv = jnp.dot(z_ref[...], w0[...], precision=_PREC,
                     preferred_element_type=jnp.float32) + b0[...]
        dv = jnp.where(dv >= 0, dv, ar[...] * dv)
        dv = jnp.dot(dv, w1[...], precision=_PREC,
                     preferred_element_type=jnp.float32) + b1r[...]
        dv = jnp.where(dv >= 0, dv, ar[...] * dv)
        l0 = jnp.dot(dv, we[...], precision=_PREC,
                     preferred_element_type=jnp.float32) + be[...] + g0r[...]
        l1 = jnp.dot(dv, wf[...], precision=_PREC,
                     preferred_element_type=jnp.float32) + bf[...] + g1r[...]
        xx = (l0 >= l1).astype(jnp.float32)
        o_ref[...] = jnp.dot(xx, gmr[...], precision=_PREC,
                             preferred_element_type=jnp.float32)

    return pl.pallas_call(
        body,
        out_shape=jax.ShapeDtypeStruct((BP, APAD), jnp.float32),
    )(z, d0W, d0b, d1W, d1b, W2e, b2e, W2f, b2f, g0, g1, Gm, av)


_IU, _JU = np.triu_indices(NMAX, k=1)
_GNP = np.zeros((PPAD, APAD), np.float32)
_GNP[np.arange(NPAIR), _IU * NMAX + _JU] = 1.0
_GNP[np.arange(NPAIR), _JU * NMAX + _IU] = 1.0


def kernel(x, edge_index, batch, params):
    p = params
    f32 = jnp.float32
    src = edge_index[0]
    dst = edge_index[1]
    eorder = jnp.argsort(dst, stable=True)
    src_sorted = src[eorder]
    dst_sorted = dst[eorder]
    bnsc = np.float32(1.0 / np.sqrt(1.0 + 1e-5))

    h = _fc0(x, p['fc0_W'], p['fc0_b'].reshape(1, H))
    for i in range(2):
        parts = _seg_sum_sc(h, src_sorted, dst_sorted)
        a1 = jnp.full((1, H), p['c%d_a1' % i], f32)
        a2 = jnp.full((1, H), p['c%d_a2' % i], f32)
        s1 = (p['c%d_bng' % i] * bnsc).reshape(1, H)
        t1 = p['c%d_bnb' % i].reshape(1, H)
        s2 = (p['bn%d_g' % i] * bnsc).reshape(1, H)
        t2 = p['bn%d_b' % i].reshape(1, H)
        h = _gin_mlp(h, parts,
                     p['c%d_W1' % i], p['c%d_b1' % i].reshape(1, H),
                     a1, s1, t1,
                     p['c%d_W2' % i], p['c%d_b2' % i].reshape(1, H),
                     a2, s2, t2)

    mu_p, lv_p = _pool_head(
        h, batch.reshape(GN, 1, RB),
        (p['bn_g'] * bnsc).reshape(1, H), p['bn_b'].reshape(1, H),
        p['fc_W'], p['fc_b'].reshape(1, H),
        p['mu_W'], p['mu_b'].reshape(1, LAT),
        p['lv_W'], p['lv_b'].reshape(1, LAT))

    u = jax.random.uniform(jax.random.key(42), (B, NPAIR, 2),
                           minval=1e-9, maxval=1.0)
    g = -jnp.log(-jnp.log(u))
    g0 = jnp.zeros((BP, PPAD), f32).at[:B, :NPAIR].set(g[:, :, 0])
    g1 = jnp.zeros((BP, PPAD), f32).at[:B, :NPAIR].set(g[:, :, 1])
    W2e = jnp.zeros((HD, PPAD), f32).at[:, :NPAIR].set(p['d2_W'][:, 0::2])
    W2f = jnp.zeros((HD, PPAD), f32).at[:, :NPAIR].set(p['d2_W'][:, 1::2])
    b2e = jnp.zeros((1, PPAD), f32).at[0, :NPAIR].set(p['d2_b'][0::2])
    b2f = jnp.zeros((1, PPAD), f32).at[0, :NPAIR].set(p['d2_b'][1::2])
    av = jnp.full((1, HD), p['dec_a'], f32)

    adjf = _decoder(mu_p, p['d0_W'], p['d0_b'].reshape(1, HD),
                    p['d1_W'], p['d1_b'].reshape(1, HD),
                    W2e, b2e, W2f, b2f, g0, g1, jnp.asarray(_GNP), av)

    mu = mu_p[:B]
    logvar = lv_p[:B]
    adj = adjf[:B, :NMAX * NMAX].reshape(B, NMAX, NMAX)
    return (adj, mu, logvar, mu)
